# baseline (speedup 1.0000x reference)
import functools

import jax
import jax.numpy as jnp
from jax import lax
from jax.experimental import pallas as pl
from jax.experimental.pallas import tpu as pltpu
from jax.experimental.pallas import tpu_sc as plsc

B, N, E = 2048, 64, 1024
C_IN, C_HID, C_OUT = 16, 32, 16
BC = 256


def _edge_count_body(src_hbm, dst_hbm, out_hbm, src_v, dst_v, flat_v, ones_v,
                     zbuf_v, acc_sh):
    wid = lax.axis_index("s") * 2 + lax.axis_index("c")

    @pl.when(wid == 0)
    def _():
        pltpu.sync_copy(src_hbm, src_v)
        pltpu.sync_copy(dst_hbm, dst_v)
        z16 = jnp.zeros((16,), jnp.float32)
        for i in range((N * N) // 16):
            zbuf_v[pl.ds(i * 16, 16)] = z16
        pltpu.sync_copy(zbuf_v, acc_sh)
        o16 = jnp.ones((16,), jnp.float32)
        for i in range(8):
            ones_v[pl.ds(i * 16, 16)] = o16
        for i in range(E // 16):
            s = src_v[pl.ds(i * 16, 16)]
            d = dst_v[pl.ds(i * 16, 16)]
            flat_v[i // 8, pl.ds((i % 8) * 16, 16)] = d * N + s
        for j in range(E // 128):
            pltpu.sync_copy(ones_v, acc_sh.at[flat_v.at[j]], add=True)
        pltpu.sync_copy(acc_sh, out_hbm)


_edge_count = functools.partial(
    pl.kernel,
    mesh=plsc.VectorSubcoreMesh(core_axis_name="c", subcore_axis_name="s"),
    out_type=jax.ShapeDtypeStruct((N * N,), jnp.float32),
    scratch_types=[
        pltpu.VMEM((E,), jnp.int32),
        pltpu.VMEM((E,), jnp.int32),
        pltpu.VMEM((E // 128, 128), jnp.int32),
        pltpu.VMEM((128,), jnp.float32),
        pltpu.VMEM((N * N,), jnp.float32),
        pltpu.VMEM_SHARED((N * N,), jnp.float32),
    ],
)(_edge_count_body)


def _gcn_body(cnt_ref, xt_ref, w1_ref, b1_ref, w2_ref, b2_ref, out_ref):
    f32 = jnp.float32
    a = cnt_ref[...]
    ii = jax.lax.broadcasted_iota(jnp.int32, (N, N), 0)
    jj = jax.lax.broadcasted_iota(jnp.int32, (N, N), 1)
    a = a + (ii == jj).astype(f32)
    deg_col = jnp.sum(a, axis=1, keepdims=True)
    ones_row = jnp.ones((1, N), dtype=f32)
    deg_row = jax.lax.dot_general(ones_row, a, (((1,), (1,)), ((), ())),
                                  preferred_element_type=f32)
    an = a * jax.lax.rsqrt(deg_col) * jax.lax.rsqrt(deg_row)
    r_row = jax.lax.dot_general(ones_row, an, (((1,), (0,)), ((), ())),
                                preferred_element_type=f32) * (1.0 / N)

    xb = xt_ref[...]
    y = jnp.dot(xb.reshape(BC * N, C_IN), w1_ref[...],
                preferred_element_type=f32)
    z = jax.lax.dot_general(y.reshape(BC, N, C_HID), an,
                            (((1,), (1,)), ((), ())),
                            preferred_element_type=f32)
    z = z + b1_ref[...].reshape(1, C_HID, 1)
    z = jnp.maximum(z, 0.0)
    p = jnp.sum(z * r_row.reshape(1, 1, N), axis=2)
    out_ref[...] = (jnp.dot(p, w2_ref[...], preferred_element_type=f32)
                    + b2_ref[...])


@jax.jit
def kernel(x, edge_index, W1, b1, W2, b2):
    edge = edge_index.astype(jnp.int32)
    cnt = _edge_count(edge[0], edge[1]).reshape(N, N)
    out = pl.pallas_call(
        _gcn_body,
        grid=(B // BC,),
        in_specs=[
            pl.BlockSpec((N, N), lambda i: (0, 0)),
            pl.BlockSpec((BC, N, C_IN), lambda i: (i, 0, 0)),
            pl.BlockSpec((C_IN, C_HID), lambda i: (0, 0)),
            pl.BlockSpec((1, C_HID), lambda i: (0, 0)),
            pl.BlockSpec((C_HID, C_OUT), lambda i: (0, 0)),
            pl.BlockSpec((1, C_OUT), lambda i: (0, 0)),
        ],
        out_specs=pl.BlockSpec((BC, C_OUT), lambda i: (i, 0)),
        out_shape=jax.ShapeDtypeStruct((B, C_OUT), jnp.float32),
        compiler_params=pltpu.CompilerParams(
            dimension_semantics=("arbitrary",)),
    )(cnt, x.astype(jnp.float32), W1.astype(jnp.float32),
      b1.reshape(1, C_HID), W2.astype(jnp.float32), b2.reshape(1, C_OUT))
    return out


# speedup vs baseline: 1.0514x; 1.0514x over previous
import functools

import jax
import jax.numpy as jnp
from jax.experimental import pallas as pl
from jax.experimental.pallas import tpu as pltpu

B, N, E = 2048, 64, 1024
C_IN, C_HID, C_OUT = 16, 32, 16
BC = 128


def _gcn_body(edge_ref, xt_ref, w1_ref, b1_ref, w2_ref, b2_ref, out_ref):
    f32 = jnp.float32
    e = edge_ref[...]
    src = e[0:1, :]
    dst = e[1:2, :]
    rows = jax.lax.broadcasted_iota(jnp.int32, (N, E), 0)
    st = (rows == src).astype(f32)
    dt = (rows == dst).astype(f32)
    a = jax.lax.dot_general(dt, st, (((1,), (1,)), ((), ())),
                            preferred_element_type=f32)
    ii = jax.lax.broadcasted_iota(jnp.int32, (N, N), 0)
    jj = jax.lax.broadcasted_iota(jnp.int32, (N, N), 1)
    a = a + (ii == jj).astype(f32)
    deg_col = jnp.sum(a, axis=1, keepdims=True)
    ones_row = jnp.ones((1, N), dtype=f32)
    deg_row = jax.lax.dot_general(ones_row, a, (((1,), (1,)), ((), ())),
                                  preferred_element_type=f32)
    an = a * jax.lax.rsqrt(deg_col) * jax.lax.rsqrt(deg_row)
    r_row = jax.lax.dot_general(ones_row, an, (((1,), (0,)), ((), ())),
                                preferred_element_type=f32) * (1.0 / N)

    xb = xt_ref[...]
    y = jnp.dot(xb.reshape(BC * N, C_IN), w1_ref[...],
                preferred_element_type=f32)
    z = jax.lax.dot_general(y.reshape(BC, N, C_HID), an,
                            (((1,), (1,)), ((), ())),
                            preferred_element_type=f32)
    z = z + b1_ref[...].reshape(1, C_HID, 1)
    z = jnp.maximum(z, 0.0)
    p = jnp.sum(z * r_row.reshape(1, 1, N), axis=2)
    out_ref[...] = (jnp.dot(p, w2_ref[...], preferred_element_type=f32)
                    + b2_ref[...])


@jax.jit
def kernel(x, edge_index, W1, b1, W2, b2):
    edge = edge_index.astype(jnp.int32)
    grid = (B // BC,)
    out = pl.pallas_call(
        _gcn_body,
        grid=grid,
        in_specs=[
            pl.BlockSpec((2, E), lambda i: (0, 0)),
            pl.BlockSpec((BC, N, C_IN), lambda i: (i, 0, 0)),
            pl.BlockSpec((C_IN, C_HID), lambda i: (0, 0)),
            pl.BlockSpec((1, C_HID), lambda i: (0, 0)),
            pl.BlockSpec((C_HID, C_OUT), lambda i: (0, 0)),
            pl.BlockSpec((1, C_OUT), lambda i: (0, 0)),
        ],
        out_specs=pl.BlockSpec((BC, C_OUT), lambda i: (i, 0)),
        out_shape=jax.ShapeDtypeStruct((B, C_OUT), jnp.float32),
        compiler_params=pltpu.CompilerParams(
            dimension_semantics=("arbitrary",)),
    )(edge, x.astype(jnp.float32), W1.astype(jnp.float32), b1.reshape(1, C_HID),
      W2.astype(jnp.float32), b2.reshape(1, C_OUT))
    return out


# speedup vs baseline: 1.2906x; 1.2275x over previous
import functools

import jax
import jax.numpy as jnp
from jax.experimental import pallas as pl
from jax.experimental.pallas import tpu as pltpu

B, N, E = 2048, 64, 1024
C_IN, C_HID, C_OUT = 16, 32, 16
BC = 256


def _gcn_body(edge_ref, xt_ref, w1_ref, b1_ref, w2_ref, b2_ref, out_ref):
    f32 = jnp.float32
    e = edge_ref[...]
    src = e[0:1, :]
    dst = e[1:2, :]
    rows = jax.lax.broadcasted_iota(jnp.int32, (N, E), 0)
    st = (rows == src).astype(f32)
    dt = (rows == dst).astype(f32)
    a = jax.lax.dot_general(dt, st, (((1,), (1,)), ((), ())),
                            preferred_element_type=f32)
    ii = jax.lax.broadcasted_iota(jnp.int32, (N, N), 0)
    jj = jax.lax.broadcasted_iota(jnp.int32, (N, N), 1)
    a = a + (ii == jj).astype(f32)
    deg_col = jnp.sum(a, axis=1, keepdims=True)
    ones_row = jnp.ones((1, N), dtype=f32)
    deg_row = jax.lax.dot_general(ones_row, a, (((1,), (1,)), ((), ())),
                                  preferred_element_type=f32)
    an = a * jax.lax.rsqrt(deg_col) * jax.lax.rsqrt(deg_row)
    r_col = jax.lax.dot_general(an, ones_row, (((0,), (1,)), ((), ())),
                                preferred_element_type=f32) * (1.0 / N)

    xt = xt_ref[...]
    y = jnp.dot(xt.reshape(N * BC, C_IN), w1_ref[...],
                preferred_element_type=f32)
    z = jax.lax.dot_general(an, y.reshape(N, BC, C_HID),
                            (((1,), (0,)), ((), ())),
                            preferred_element_type=f32)
    z = z + b1_ref[...].reshape(1, 1, C_HID)
    z = jnp.maximum(z, 0.0)
    p = jnp.sum(z * r_col.reshape(N, 1, 1), axis=0)
    out_ref[...] = (jnp.dot(p, w2_ref[...], preferred_element_type=f32)
                    + b2_ref[...])


@jax.jit
def kernel(x, edge_index, W1, b1, W2, b2):
    xt = jnp.transpose(x.astype(jnp.float32), (1, 0, 2))
    edge = edge_index.astype(jnp.int32)
    grid = (B // BC,)
    out = pl.pallas_call(
        _gcn_body,
        grid=grid,
        in_specs=[
            pl.BlockSpec((2, E), lambda i: (0, 0)),
            pl.BlockSpec((N, BC, C_IN), lambda i: (0, i, 0)),
            pl.BlockSpec((C_IN, C_HID), lambda i: (0, 0)),
            pl.BlockSpec((1, C_HID), lambda i: (0, 0)),
            pl.BlockSpec((C_HID, C_OUT), lambda i: (0, 0)),
            pl.BlockSpec((1, C_OUT), lambda i: (0, 0)),
        ],
        out_specs=pl.BlockSpec((BC, C_OUT), lambda i: (i, 0)),
        out_shape=jax.ShapeDtypeStruct((B, C_OUT), jnp.float32),
        compiler_params=pltpu.CompilerParams(
            dimension_semantics=("arbitrary",)),
    )(edge, xt, W1.astype(jnp.float32), b1.reshape(1, C_HID),
      W2.astype(jnp.float32), b2.reshape(1, C_OUT))
    return out
